# baseline (speedup 1.0000x reference)
import functools

import jax
import jax.numpy as jnp
from jax import lax
from jax.experimental import pallas as pl
from jax.experimental.pallas import tpu as pltpu
from jax.experimental.pallas import tpu_sc as plsc

BATCH = 16384
DIM = 128
LANES = 16
CHUNK = 128


def _sc_body(z_hbm, j_hbm, ep_hbm, out_hbm,
             idx_c, idxp1_c, lo_c, hi_c, z_c, o_c, sem):
    nc = 2
    wid = lax.axis_index("s") * nc + lax.axis_index("c")
    rows_per_w = BATCH // (nc * 16)
    n_chunks = rows_per_w // CHUNK
    base = wid * rows_per_w

    for c in range(n_chunks):
        row0 = base + c * CHUNK
        pltpu.sync_copy(j_hbm.at[pl.ds(row0, CHUNK)], idx_c)
        for v in range(CHUNK // LANES):
            s = pl.ds(v * LANES, LANES)
            idxp1_c[s] = idx_c[s] + 1
        cp_lo = pltpu.async_copy(ep_hbm.at[idx_c], lo_c, sem)
        cp_hi = pltpu.async_copy(ep_hbm.at[idxp1_c], hi_c, sem)
        pltpu.sync_copy(z_hbm.at[pl.ds(row0, CHUNK), :], z_c)
        cp_lo.wait()
        cp_hi.wait()

        def row_body(r, _):
            ridx = jnp.full((LANES,), r, dtype=jnp.int32)
            lo_v = plsc.load_gather(lo_c, [ridx])
            hi_v = plsc.load_gather(hi_c, [ridx])
            sc_v = hi_v - lo_v
            for v in range(DIM // LANES):
                s = pl.ds(v * LANES, LANES)
                o_c[r, s] = z_c[r, s] * sc_v + lo_v
            return 0

        lax.fori_loop(0, CHUNK, row_body, 0)
        pltpu.sync_copy(o_c, out_hbm.at[pl.ds(row0, CHUNK), :])


@jax.jit
def _shift_mapper_sc(z, j_flat, endpoints):
    mesh = plsc.VectorSubcoreMesh(core_axis_name="c", subcore_axis_name="s")
    kfn = pl.kernel(
        _sc_body,
        mesh=mesh,
        out_type=jax.ShapeDtypeStruct((BATCH, DIM), jnp.float32),
        scratch_types=[
            pltpu.VMEM((CHUNK,), jnp.int32),
            pltpu.VMEM((CHUNK,), jnp.int32),
            pltpu.VMEM((CHUNK,), jnp.float32),
            pltpu.VMEM((CHUNK,), jnp.float32),
            pltpu.VMEM((CHUNK, DIM), jnp.float32),
            pltpu.VMEM((CHUNK, DIM), jnp.float32),
            pltpu.SemaphoreType.DMA,
        ],
        compiler_params=pltpu.CompilerParams(needs_layout_passes=False),
    )
    return kfn(z, j_flat, endpoints)


def kernel(z, j, endpoints):
    j_flat = j.reshape(-1).astype(jnp.int32)
    return _shift_mapper_sc(z, j_flat, endpoints)


# speedup vs baseline: 1.7512x; 1.7512x over previous
import jax
import jax.numpy as jnp
from jax import lax
from jax.experimental import pallas as pl
from jax.experimental.pallas import tpu as pltpu
from jax.experimental.pallas import tpu_sc as plsc

BATCH = 16384
DIM = 128
LANES = 16
CHUNK = 128
N_WORKERS = 32
ROWS_PER_W = BATCH // N_WORKERS
N_CHUNKS = ROWS_PER_W // CHUNK


def _sc_body(z_hbm, j_hbm, ep_hbm, out_hbm,
             idx_v, idxp1_v, lo_v, hi_v,
             z_b0, z_b1, o_b0, o_b1,
             sem_g, sem_z0, sem_z1, sem_o0, sem_o1):
    wid = lax.axis_index("s") * 2 + lax.axis_index("c")
    base = wid * ROWS_PER_W
    z_b = [z_b0, z_b1]
    o_b = [o_b0, o_b1]
    sem_z = [sem_z0, sem_z1]
    sem_o = [sem_o0, sem_o1]

    gathers = []
    for c in range(N_CHUNKS):
        pltpu.sync_copy(j_hbm.at[pl.ds(base + c * CHUNK, CHUNK)], idx_v.at[c])
        for v in range(CHUNK // LANES):
            s = pl.ds(v * LANES, LANES)
            idxp1_v[c, s] = idx_v[c, s] + 1
        gathers.append(pltpu.async_copy(ep_hbm.at[idx_v.at[c]], lo_v.at[c], sem_g))
        gathers.append(pltpu.async_copy(ep_hbm.at[idxp1_v.at[c]], hi_v.at[c], sem_g))

    z_cp = {}
    for c in range(2):
        z_cp[c] = pltpu.async_copy(
            z_hbm.at[pl.ds(base + c * CHUNK, CHUNK), :], z_b[c], sem_z[c])
    for g in gathers:
        g.wait()

    o_cp = {}
    for c in range(N_CHUNKS):
        b = c % 2
        z_cp[c].wait()
        if c >= 2:
            o_cp[c - 2].wait()

        def grp_body(gi, _):
            lo_vec = lo_v[c, pl.ds(gi * LANES, LANES)]
            hi_vec = hi_v[c, pl.ds(gi * LANES, LANES)]
            sc_vec = hi_vec - lo_vec
            row0 = gi * LANES
            for r in range(LANES):
                lo_s = lo_vec[r]
                sc_s = sc_vec[r]
                for v in range(DIM // LANES):
                    s = pl.ds(v * LANES, LANES)
                    o_b[b][row0 + r, s] = z_b[b][row0 + r, s] * sc_s + lo_s
            return 0

        lax.fori_loop(0, CHUNK // LANES, grp_body, 0)

        o_cp[c] = pltpu.async_copy(
            o_b[b], out_hbm.at[pl.ds(base + c * CHUNK, CHUNK), :], sem_o[b])
        if c + 2 < N_CHUNKS:
            z_cp[c + 2] = pltpu.async_copy(
                z_hbm.at[pl.ds(base + (c + 2) * CHUNK, CHUNK), :], z_b[b],
                sem_z[b])
    o_cp[N_CHUNKS - 2].wait()
    o_cp[N_CHUNKS - 1].wait()


@jax.jit
def _shift_mapper_sc(z, j_flat, endpoints):
    mesh = plsc.VectorSubcoreMesh(core_axis_name="c", subcore_axis_name="s")
    kfn = pl.kernel(
        _sc_body,
        mesh=mesh,
        out_type=jax.ShapeDtypeStruct((BATCH, DIM), jnp.float32),
        scratch_types=[
            pltpu.VMEM((N_CHUNKS, CHUNK), jnp.int32),
            pltpu.VMEM((N_CHUNKS, CHUNK), jnp.int32),
            pltpu.VMEM((N_CHUNKS, CHUNK), jnp.float32),
            pltpu.VMEM((N_CHUNKS, CHUNK), jnp.float32),
            pltpu.VMEM((CHUNK, DIM), jnp.float32),
            pltpu.VMEM((CHUNK, DIM), jnp.float32),
            pltpu.VMEM((CHUNK, DIM), jnp.float32),
            pltpu.VMEM((CHUNK, DIM), jnp.float32),
            pltpu.SemaphoreType.DMA,
            pltpu.SemaphoreType.DMA,
            pltpu.SemaphoreType.DMA,
            pltpu.SemaphoreType.DMA,
            pltpu.SemaphoreType.DMA,
        ],
        compiler_params=pltpu.CompilerParams(needs_layout_passes=False),
    )
    return kfn(z, j_flat, endpoints)


def kernel(z, j, endpoints):
    j_flat = j.reshape(-1).astype(jnp.int32)
    return _shift_mapper_sc(z, j_flat, endpoints)
